# baseline (speedup 1.0000x reference)
import functools

import jax
import jax.numpy as jnp
from jax.experimental import pallas as pl
from jax.experimental.pallas import tpu as pltpu

B = 16384
IN_DIM = 768
L = 256

TILE_B = 4096


def _vq_kernel(x_ref, w_ref, b_ref, e0_ref, e1_ref,
               idx_ref, quant_ref, ze_ref):
    z = jnp.dot(x_ref[...], w_ref[...], preferred_element_type=jnp.float32)
    z = z + b_ref[...]
    e0 = e0_ref[...]
    e1 = e1_ref[...]
    d0 = (z - e0) ** 2
    d1 = (z - e1) ** 2
    take1 = d1 < d0
    idx_ref[...] = take1.astype(jnp.int32)
    quant_ref[...] = jnp.where(take1, e1, e0)
    ze_ref[...] = z


@jax.jit
def kernel(x, embedding, W, b):
    e0 = embedding[:, 0, 0].reshape(1, L)
    e1 = embedding[:, 1, 0].reshape(1, L)
    b2 = b.reshape(1, L)

    grid = (B // TILE_B,)
    out_shapes = (
        jax.ShapeDtypeStruct((B, L), jnp.int32),
        jax.ShapeDtypeStruct((B, L), jnp.float32),
        jax.ShapeDtypeStruct((B, L), jnp.float32),
    )
    row_spec = pl.BlockSpec((TILE_B, L), lambda i: (i, 0))
    indices, quantized, z_e = pl.pallas_call(
        _vq_kernel,
        grid=grid,
        in_specs=[
            pl.BlockSpec((TILE_B, IN_DIM), lambda i: (i, 0)),
            pl.BlockSpec((IN_DIM, L), lambda i: (0, 0)),
            pl.BlockSpec((1, L), lambda i: (0, 0)),
            pl.BlockSpec((1, L), lambda i: (0, 0)),
            pl.BlockSpec((1, L), lambda i: (0, 0)),
        ],
        out_specs=(row_spec, row_spec, row_spec),
        out_shape=out_shapes,
        compiler_params=pltpu.CompilerParams(
            dimension_semantics=("parallel",),
            vmem_limit_bytes=128 * 1024 * 1024,
        ),
    )(x, W, b2, e0, e1)
    return (indices, embedding, quantized, z_e)


# speedup vs baseline: 1.0230x; 1.0230x over previous
import functools

import jax
import jax.numpy as jnp
from jax.experimental import pallas as pl
from jax.experimental.pallas import tpu as pltpu

B = 16384
IN_DIM = 768
L = 256

TILE_B = 4096


def _vq_kernel(x_ref, w_ref, b_ref, e0_ref, e1_ref,
               idx_ref, quant_ref, ze_ref):
    z = jnp.dot(x_ref[...], w_ref[...], preferred_element_type=jnp.float32)
    z = z + b_ref[...]
    e0 = e0_ref[...]
    e1 = e1_ref[...]
    d0 = (z - e0) ** 2
    d1 = (z - e1) ** 2
    take1 = d1 < d0
    idx_ref[...] = take1.astype(jnp.int32)
    quant_ref[...] = jnp.where(take1, e1, e0)
    ze_ref[...] = z


@jax.jit
def kernel(x, embedding, W, b):
    e0 = embedding[:, 0, 0].reshape(1, L)
    e1 = embedding[:, 1, 0].reshape(1, L)
    b2 = b.reshape(1, L)

    grid = (B // TILE_B,)
    out_shapes = (
        jax.ShapeDtypeStruct((B, L), jnp.int32),
        jax.ShapeDtypeStruct((B, L), jnp.float32),
        jax.ShapeDtypeStruct((B, L), jnp.float32),
    )
    row_spec = pl.BlockSpec((TILE_B, L), lambda i: (i, 0))
    indices, quantized, z_e = pl.pallas_call(
        _vq_kernel,
        grid=grid,
        in_specs=[
            pl.BlockSpec((TILE_B, IN_DIM), lambda i: (i, 0)),
            pl.BlockSpec((IN_DIM, L), lambda i: (0, 0)),
            pl.BlockSpec((1, L), lambda i: (0, 0)),
            pl.BlockSpec((1, L), lambda i: (0, 0)),
            pl.BlockSpec((1, L), lambda i: (0, 0)),
        ],
        out_specs=(row_spec, row_spec, row_spec),
        out_shape=out_shapes,
        compiler_params=pltpu.CompilerParams(
            dimension_semantics=("parallel",),
        ),
    )(x, W, b2, e0, e1)
    return (indices, embedding, quantized, z_e)
